# baseline (speedup 1.0000x reference)
import functools

import jax
import jax.numpy as jnp
import numpy as np
from jax import lax
from jax.experimental import pallas as pl
from jax.experimental.pallas import tpu as pltpu
from jax.experimental.pallas import tpu_sc as plsc

N_BATCH = 2
NUM_RAYS = 4096
DEPTH_RES = 48
N_IMPORTANCE = 48
RAY_START = 2.25
RAY_END = 3.3
C_FEAT = 32
HW = 256
HIDDEN = 64

NR = N_BATCH * NUM_RAYS
S = DEPTH_RES
M = NR * S
NTAP = 12
PLANE = HW * HW
S2 = 2 * S

_SU47 = np.triu(np.ones((47, 47), np.float32), 1)
_SU95 = np.triu(np.ones((95, 95), np.float32), 1)
_U45 = np.triu(np.ones((45, 45), np.float32), 0)

_RB = 512


def _softplus(x):
    return jnp.maximum(x, 0.0) + jnp.log1p(jnp.exp(-jnp.abs(x)))


def _taps(gx, gy, base, idx_out, w_out, p):
    x = (gx + 1.0) * (HW / 2.0) - 0.5
    y = (gy + 1.0) * (HW / 2.0) - 0.5
    x0 = jnp.floor(x)
    y0 = jnp.floor(y)
    wx1 = x - x0
    wx0 = 1.0 - wx1
    wy1 = y - y0
    wy0 = 1.0 - wy1
    taps = ((x0, y0, wx0 * wy0), (x0 + 1.0, y0, wx1 * wy0),
            (x0, y0 + 1.0, wx0 * wy1), (x0 + 1.0, y0 + 1.0, wx1 * wy1))
    for k, (xf, yf, wgt) in enumerate(taps):
        valid = ((xf >= 0) & (xf < HW) & (yf >= 0) & (yf < HW)).astype(jnp.float32)
        ix = jnp.clip(xf, 0.0, HW - 1.0).astype(jnp.int32)
        iy = jnp.clip(yf, 0.0, HW - 1.0).astype(jnp.int32)
        t = p * 4 + k
        idx_out[:, t * S:(t + 1) * S] = base + iy * HW + ix
        w_out[:, t * S:(t + 1) * S] = wgt * valid * jnp.float32(1.0 / 3.0)


def _emit_taps(ox, oy, oz, dx, dy, dz, depths, pid, idx_out, w_out):
    n = (pid * _RB) // NUM_RAYS
    base0 = n * 3 * PLANE
    cx = 2.0 * (ox + depths * dx)
    cy = 2.0 * (oy + depths * dy)
    cz = 2.0 * (oz + depths * dz)
    _taps(cx, cy, base0 + 0 * PLANE, idx_out, w_out, 0)
    _taps(cx, cz, base0 + 1 * PLANE, idx_out, w_out, 1)
    _taps(cz, cx, base0 + 2 * PLANE, idx_out, w_out, 2)


def _k1_body(ro_ref, rd_ref, dc_ref, idx_out, w_out):
    pid = pl.program_id(0)
    dc = dc_ref[0:1, :]
    _emit_taps(ro_ref[:, 0:1], ro_ref[:, 1:2], ro_ref[:, 2:3],
               rd_ref[:, 0:1], rd_ref[:, 1:2], rd_ref[:, 2:3],
               dc, pid, idx_out, w_out)


def _k1_call(ro, rd, dc_row):
    grid = NR // _RB
    return pl.pallas_call(
        _k1_body,
        grid=(grid,),
        in_specs=[pl.BlockSpec((_RB, 3), lambda i: (i, 0)),
                  pl.BlockSpec((_RB, 3), lambda i: (i, 0)),
                  pl.BlockSpec((1, S), lambda i: (0, 0))],
        out_specs=[pl.BlockSpec((_RB, NTAP * S), lambda i: (i, 0)),
                   pl.BlockSpec((_RB, NTAP * S), lambda i: (i, 0))],
        out_shape=[jax.ShapeDtypeStruct((NR, NTAP * S), jnp.int32),
                   jax.ShapeDtypeStruct((NR, NTAP * S), jnp.float32)],
    )(ro, rd, dc_row)


_MB = 4096


def _k2a_body(x_ref, w1_ref, b1_ref, w2_ref, b2_ref, o_ref):
    h = jnp.dot(x_ref[:, :], w1_ref[:, :], preferred_element_type=jnp.float32)
    h = _softplus(h + b1_ref[0:1, :])
    o = jnp.dot(h, w2_ref[:, :], preferred_element_type=jnp.float32) + b2_ref[0:1, :]
    sigma = o[:, 0:1]
    rgb = (1.0 / (1.0 + jnp.exp(-o[:, 1:4]))) * 1.002 - 0.001
    o_ref[:, :] = jnp.concatenate([sigma, rgb], axis=1)


def _k2a_call(feats, w1, b1r, w2, b2r):
    grid = M // _MB
    return pl.pallas_call(
        _k2a_body,
        grid=(grid,),
        in_specs=[pl.BlockSpec((_MB, C_FEAT), lambda i: (i, 0)),
                  pl.BlockSpec((C_FEAT, HIDDEN), lambda i: (0, 0)),
                  pl.BlockSpec((1, HIDDEN), lambda i: (0, 0)),
                  pl.BlockSpec((HIDDEN, 4), lambda i: (0, 0)),
                  pl.BlockSpec((1, 4), lambda i: (0, 0))],
        out_specs=pl.BlockSpec((_MB, 4), lambda i: (i, 0)),
        out_shape=jax.ShapeDtypeStruct((M, 4), jnp.float32),
    )(feats, w1, b1r, w2, b2r)


def _k2b_body(sig_ref, ro_ref, rd_ref, dc_ref, u_ref, su47_ref, u45_ref,
              df_out, idx_out, w_out, posc_out, posf_out):
    pid = pl.program_id(0)
    dc = dc_ref[0:1, :]
    sig = sig_ref[:, :]
    deltas = dc[:, 1:] - dc[:, :-1]
    dm = 0.5 * (sig[:, :-1] + sig[:, 1:])
    dd = _softplus(dm - 1.0) * deltas
    e = jnp.exp(-dd)
    alpha = 1.0 - e
    logt = jnp.log(e + 1e-10)
    texcl = jnp.exp(jnp.dot(logt, su47_ref[:, :], preferred_element_type=jnp.float32))
    w = alpha * texcl

    ninf = jnp.float32(-jnp.inf)
    left = jnp.concatenate([jnp.full((w.shape[0], 1), ninf), w], axis=1)
    right = jnp.concatenate([w, jnp.full((w.shape[0], 1), ninf)], axis=1)
    wmax = jnp.maximum(left, right)
    wavg = 0.5 * (wmax[:, :-1] + wmax[:, 1:]) + 0.01
    wpdf = wavg[:, 1:46] + 1e-5
    pdf = wpdf / jnp.sum(wpdf, axis=1, keepdims=True)
    cdf = jnp.dot(pdf, u45_ref[:, :], preferred_element_type=jnp.float32)
    u = u_ref[0:1, :]

    zmid = 0.5 * (dc[:, :-1] + dc[:, 1:])
    zero = jnp.zeros(sig.shape, jnp.float32)
    cdf_g0 = zero
    bins_g0 = zero + zmid[:, 0:1]
    cdf_g1 = zero + cdf[:, 44:45]
    bins_g1 = zero + zmid[:, 45:46]
    for j in range(45):
        m = cdf[:, j:j + 1] <= u
        cdf_g0 = jnp.where(m, cdf[:, j:j + 1], cdf_g0)
        bins_g0 = jnp.where(m, zmid[:, j + 1:j + 2], bins_g0)
    for j in range(44, -1, -1):
        m = cdf[:, j:j + 1] > u
        cdf_g1 = jnp.where(m, cdf[:, j:j + 1], cdf_g1)
        bins_g1 = jnp.where(m, zmid[:, j + 1:j + 2], bins_g1)
    denom = cdf_g1 - cdf_g0
    denom = jnp.where(denom < 1e-5, 1.0, denom)
    df = bins_g0 + (u - cdf_g0) / denom * (bins_g1 - bins_g0)
    df_out[:, :] = df

    iota = lax.broadcasted_iota(jnp.int32, sig.shape, 1)
    pc = iota
    pf = iota
    for j in range(S):
        pc = pc + (df[:, j:j + 1] < dc).astype(jnp.int32)
        pf = pf + (dc[:, j:j + 1] <= df).astype(jnp.int32)
    posc_out[:, :] = pc
    posf_out[:, :] = pf

    _emit_taps(ro_ref[:, 0:1], ro_ref[:, 1:2], ro_ref[:, 2:3],
               rd_ref[:, 0:1], rd_ref[:, 1:2], rd_ref[:, 2:3],
               df, pid, idx_out, w_out)


def _k2b_call(sig, ro, rd, dc_row, u_row):
    grid = NR // _RB
    return pl.pallas_call(
        _k2b_body,
        grid=(grid,),
        in_specs=[pl.BlockSpec((_RB, S), lambda i: (i, 0)),
                  pl.BlockSpec((_RB, 3), lambda i: (i, 0)),
                  pl.BlockSpec((_RB, 3), lambda i: (i, 0)),
                  pl.BlockSpec((1, S), lambda i: (0, 0)),
                  pl.BlockSpec((1, S), lambda i: (0, 0)),
                  pl.BlockSpec((47, 47), lambda i: (0, 0)),
                  pl.BlockSpec((45, 45), lambda i: (0, 0))],
        out_specs=[pl.BlockSpec((_RB, S), lambda i: (i, 0)),
                   pl.BlockSpec((_RB, NTAP * S), lambda i: (i, 0)),
                   pl.BlockSpec((_RB, NTAP * S), lambda i: (i, 0)),
                   pl.BlockSpec((_RB, S), lambda i: (i, 0)),
                   pl.BlockSpec((_RB, S), lambda i: (i, 0))],
        out_shape=[jax.ShapeDtypeStruct((NR, S), jnp.float32),
                   jax.ShapeDtypeStruct((NR, NTAP * S), jnp.int32),
                   jax.ShapeDtypeStruct((NR, NTAP * S), jnp.float32),
                   jax.ShapeDtypeStruct((NR, S), jnp.int32),
                   jax.ShapeDtypeStruct((NR, S), jnp.int32)],
    )(sig, ro, rd, dc_row, u_row, jnp.asarray(_SU47), jnp.asarray(_U45))


def _k4c_body(d_ref, r_ref, g_ref, b_ref, s_ref, dc_ref, su95_ref,
              rgb_out, dep_out, ws_out):
    d = d_ref[:, :]
    sg = s_ref[:, :]
    deltas = d[:, 1:] - d[:, :-1]
    dmid = 0.5 * (d[:, :-1] + d[:, 1:])
    smid = 0.5 * (sg[:, :-1] + sg[:, 1:])
    dd = _softplus(smid - 1.0) * deltas
    e = jnp.exp(-dd)
    alpha = 1.0 - e
    logt = jnp.log(e + 1e-10)
    texcl = jnp.exp(jnp.dot(logt, su95_ref[:, :], preferred_element_type=jnp.float32))
    w = alpha * texcl
    wtot = jnp.sum(w, axis=1, keepdims=True)
    outs = []
    for c_ref in (r_ref, g_ref, b_ref):
        c = c_ref[:, :]
        cmid = 0.5 * (c[:, :-1] + c[:, 1:])
        outs.append(jnp.sum(w * cmid, axis=1, keepdims=True))
    rgb = jnp.concatenate(outs, axis=1) * 2.0 - 1.0
    dep = jnp.sum(w * dmid, axis=1, keepdims=True) / wtot
    dep = jnp.where(dep != dep, jnp.float32(jnp.inf), dep)
    dmin = dc_ref[0:1, 0:1]
    dmax = dc_ref[0:1, S - 1:S]
    dep = jnp.minimum(jnp.maximum(dep, dmin), dmax)
    rgb_out[:, :] = rgb
    dep_out[:, :] = dep
    ws_out[:, :] = wtot


def _k4c_call(sd, sr, sg, sb, ss, dc_row):
    grid = NR // _RB
    return pl.pallas_call(
        _k4c_body,
        grid=(grid,),
        in_specs=[pl.BlockSpec((_RB, S2), lambda i: (i, 0)) for _ in range(5)] +
                 [pl.BlockSpec((1, S), lambda i: (0, 0)),
                  pl.BlockSpec((95, 95), lambda i: (0, 0))],
        out_specs=[pl.BlockSpec((_RB, 3), lambda i: (i, 0)),
                   pl.BlockSpec((_RB, 1), lambda i: (i, 0)),
                   pl.BlockSpec((_RB, 1), lambda i: (i, 0))],
        out_shape=[jax.ShapeDtypeStruct((NR, 3), jnp.float32),
                   jax.ShapeDtypeStruct((NR, 1), jnp.float32),
                   jax.ShapeDtypeStruct((NR, 1), jnp.float32)],
    )(sd, sr, sg, sb, ss, dc_row, jnp.asarray(_SU95))


_G = 96
_IDXC = _G * NTAP
_NW = 32
_SAMP_W = M // _NW
_CHUNKS = _SAMP_W // _G


def _scg_body(table_hbm, idx_hbm, w_hbm, out_hbm,
              idx_v0, idx_v1, w_v0, w_v1, rows_v0, rows_v1, out_v0, out_v1,
              sem_io, sem_g, sem_out):
    nc = 2
    wid = lax.axis_index("s") * nc + lax.axis_index("c")
    base_s = wid * _SAMP_W
    nsub = _IDXC // 128
    bufs = ((idx_v0, w_v0, rows_v0, out_v0), (idx_v1, w_v1, rows_v1, out_v1))

    def io_start(c, b):
        ib = (base_s + c * _G) * NTAP
        idx_v, w_v = bufs[b][0], bufs[b][1]
        pltpu.async_copy(idx_hbm.at[pl.ds(ib, _IDXC)], idx_v, sem_io)
        pltpu.async_copy(w_hbm.at[pl.ds(ib, _IDXC)], w_v, sem_io)

    def io_wait(c, b):
        ib = (base_s + c * _G) * NTAP
        idx_v, w_v = bufs[b][0], bufs[b][1]
        pltpu.make_async_copy(idx_hbm.at[pl.ds(ib, _IDXC)], idx_v, sem_io).wait()
        pltpu.make_async_copy(w_hbm.at[pl.ds(ib, _IDXC)], w_v, sem_io).wait()

    def g_start(b):
        idx_v, rows_v = bufs[b][0], bufs[b][2]
        for j in range(nsub):
            pltpu.async_copy(table_hbm.at[idx_v.at[pl.ds(j * 128, 128)]],
                             rows_v.at[pl.ds(j * 128, 128)], sem_g)

    def g_wait(b):
        idx_v, rows_v = bufs[b][0], bufs[b][2]
        for j in range(nsub):
            pltpu.make_async_copy(table_hbm.at[idx_v.at[pl.ds(j * 128, 128)]],
                                  rows_v.at[pl.ds(j * 128, 128)], sem_g).wait()

    def out_start(c, b):
        pltpu.async_copy(bufs[b][3], out_hbm.at[pl.ds(base_s + c * _G, _G), :],
                         sem_out)

    def out_wait(c, b):
        pltpu.make_async_copy(bufs[b][3],
                              out_hbm.at[pl.ds(base_s + c * _G, _G), :],
                              sem_out).wait()

    def compute(b):
        w_v, rows_v, out_v = bufs[b][1], bufs[b][2], bufs[b][3]
        iota = lax.iota(jnp.int32, 16)
        for lr in range(2):
            for g16 in range(3):
                srow = lr * (NTAP * S) + g16 * 16
                svec = jnp.full((16,), lr * S + g16 * 16, jnp.int32) + iota
                wts = [w_v[pl.ds(srow + t * S, 16)] for t in range(NTAP)]
                rvecs = [jnp.full((16,), srow + t * S, jnp.int32) + iota
                         for t in range(NTAP)]

                def chan(ch, _):
                    cvec = jnp.full((16,), ch, jnp.int32)
                    acc = jnp.zeros((16,), jnp.float32)
                    for t in range(NTAP):
                        v = plsc.load_gather(rows_v, [rvecs[t], cvec])
                        acc = acc + wts[t] * v
                    plsc.store_scatter(out_v, [svec, cvec], acc)
                    return _

                lax.fori_loop(0, C_FEAT, chan, 0)

    io_start(0, 0)
    io_wait(0, 0)
    g_start(0)
    io_start(1, 1)

    def step(cc, _):
        for b in (0, 1):
            c = cc * 2 + b
            io_wait(c + 1, 1 - b)
            g_start(1 - b)
            g_wait(b)
            lax.cond(c >= 2, lambda: out_wait(c - 2, b), lambda: None)
            compute(b)
            lax.cond(c + 2 < _CHUNKS, lambda: io_start(c + 2, b), lambda: None)
            out_start(c, b)
        return _

    lax.fori_loop(0, (_CHUNKS - 2) // 2, step, 0)
    c0 = _CHUNKS - 2
    for b in (0, 1):
        c = c0 + b
        if c + 1 < _CHUNKS:
            io_wait(c + 1, 1 - b)
            g_start(1 - b)
        g_wait(b)
        out_wait(c - 2, b)
        compute(b)
        out_start(c, b)
    out_wait(_CHUNKS - 2, 0)
    out_wait(_CHUNKS - 1, 1)


def _scg_call(table, idx_flat, w_flat):
    mesh = plsc.VectorSubcoreMesh(core_axis_name="c", subcore_axis_name="s")
    f = functools.partial(
        pl.kernel, mesh=mesh,
        compiler_params=pltpu.CompilerParams(use_tc_tiling_on_sc=False,
                                             needs_layout_passes=False),
        out_type=jax.ShapeDtypeStruct((M, C_FEAT), jnp.float32),
        scratch_types=[pltpu.VMEM((_IDXC,), jnp.int32),
                       pltpu.VMEM((_IDXC,), jnp.int32),
                       pltpu.VMEM((_IDXC,), jnp.float32),
                       pltpu.VMEM((_IDXC,), jnp.float32),
                       pltpu.VMEM((_IDXC, C_FEAT), jnp.float32),
                       pltpu.VMEM((_IDXC, C_FEAT), jnp.float32),
                       pltpu.VMEM((_G, C_FEAT), jnp.float32),
                       pltpu.VMEM((_G, C_FEAT), jnp.float32),
                       pltpu.SemaphoreType.DMA,
                       pltpu.SemaphoreType.DMA,
                       pltpu.SemaphoreType.DMA],
    )(_scg_body)
    return f(table, idx_flat, w_flat)


_CR = 32
_RAYS_W = NR // _NW
_MCHUNKS = _RAYS_W // _CR


def _scm_body(posc_hbm, posf_hbm, df_hbm, dc_hbm,
              sc_hbm, rc_hbm, gc_hbm, bc_hbm,
              sf_hbm, rf_hbm, gf_hbm, bf_hbm,
              od_hbm, or_hbm, og_hbm, ob_hbm, os_hbm,
              posc_v, posf_v, df_v, dc_v,
              sc_v, rc_v, gc_v, bc_v, sf_v, rf_v, gf_v, bf_v,
              od_v, or_v, og_v, ob_v, os_v):
    nc = 2
    wid = lax.axis_index("s") * nc + lax.axis_index("c")
    pltpu.sync_copy(dc_hbm, dc_v)

    def chunk(c, _):
        rbase = wid * _RAYS_W + c * _CR
        sb = rbase * S
        ns = _CR * S
        pltpu.sync_copy(posc_hbm.at[pl.ds(sb, ns)], posc_v)
        pltpu.sync_copy(posf_hbm.at[pl.ds(sb, ns)], posf_v)
        pltpu.sync_copy(df_hbm.at[pl.ds(sb, ns)], df_v)
        for src, dst in ((sc_hbm, sc_v), (rc_hbm, rc_v), (gc_hbm, gc_v),
                         (bc_hbm, bc_v), (sf_hbm, sf_v), (rf_hbm, rf_v),
                         (gf_hbm, gf_v), (bf_hbm, bf_v)):
            pltpu.sync_copy(src.at[pl.ds(sb, ns)], dst)

        def ray(r, _):
            rb96 = r * S2
            for g in range(S // 16):
                off = r * S + g * 16
                pc = posc_v[pl.ds(off, 16)] + rb96
                pf = posf_v[pl.ds(off, 16)] + rb96
                plsc.store_scatter(od_v, [pc], dc_v[pl.ds(g * 16, 16)])
                plsc.store_scatter(os_v, [pc], sc_v[pl.ds(off, 16)])
                plsc.store_scatter(or_v, [pc], rc_v[pl.ds(off, 16)])
                plsc.store_scatter(og_v, [pc], gc_v[pl.ds(off, 16)])
                plsc.store_scatter(ob_v, [pc], bc_v[pl.ds(off, 16)])
                plsc.store_scatter(od_v, [pf], df_v[pl.ds(off, 16)])
                plsc.store_scatter(os_v, [pf], sf_v[pl.ds(off, 16)])
                plsc.store_scatter(or_v, [pf], rf_v[pl.ds(off, 16)])
                plsc.store_scatter(og_v, [pf], gf_v[pl.ds(off, 16)])
                plsc.store_scatter(ob_v, [pf], bf_v[pl.ds(off, 16)])
            return _

        lax.fori_loop(0, _CR, ray, 0)
        ob = rbase * S2
        nso = _CR * S2
        pltpu.sync_copy(od_v, od_hbm.at[pl.ds(ob, nso)])
        pltpu.sync_copy(or_v, or_hbm.at[pl.ds(ob, nso)])
        pltpu.sync_copy(og_v, og_hbm.at[pl.ds(ob, nso)])
        pltpu.sync_copy(ob_v, ob_hbm.at[pl.ds(ob, nso)])
        pltpu.sync_copy(os_v, os_hbm.at[pl.ds(ob, nso)])
        return _

    lax.fori_loop(0, _MCHUNKS, chunk, 0)


def _scm_call(posc, posf, df, dc48, sc, rc, gc, bc, sf, rf, gf, bf):
    mesh = plsc.VectorSubcoreMesh(core_axis_name="c", subcore_axis_name="s")
    ns = _CR * S
    nso = _CR * S2
    f = functools.partial(
        pl.kernel, mesh=mesh,
        compiler_params=pltpu.CompilerParams(use_tc_tiling_on_sc=False,
                                             needs_layout_passes=False),
        out_type=[jax.ShapeDtypeStruct((NR * S2,), jnp.float32) for _ in range(5)],
        scratch_types=[pltpu.VMEM((ns,), jnp.int32),
                       pltpu.VMEM((ns,), jnp.int32),
                       pltpu.VMEM((ns,), jnp.float32),
                       pltpu.VMEM((S,), jnp.float32)] +
                      [pltpu.VMEM((ns,), jnp.float32) for _ in range(8)] +
                      [pltpu.VMEM((nso,), jnp.float32) for _ in range(5)],
    )(_scm_body)
    return f(posc, posf, df, dc48, sc, rc, gc, bc, sf, rf, gf, bf)


def kernel(planes, ray_origins, ray_directions, w1, b1, w2, b2):
    table = planes.transpose(0, 1, 3, 4, 2).reshape(N_BATCH * 3 * PLANE, C_FEAT)
    delta = (RAY_END - RAY_START) / (DEPTH_RES - 1)
    dc_row = (jnp.linspace(RAY_START, RAY_END, S, dtype=jnp.float32)
              + jnp.float32(0.5 * delta)).reshape(1, S)
    u_row = jnp.linspace(0.0, 1.0, N_IMPORTANCE, dtype=jnp.float32).reshape(1, S)
    ro = ray_origins.reshape(NR, 3)
    rd = ray_directions.reshape(NR, 3)
    b1r = b1.reshape(1, HIDDEN)
    b2r = b2.reshape(1, 4)

    idx_c, w_c = _k1_call(ro, rd, dc_row)
    feats_c = _scg_call(table, idx_c.reshape(-1), w_c.reshape(-1))
    o_c = _k2a_call(feats_c, w1, b1r, w2, b2r)

    sig_c = o_c[:, 0].reshape(NR, S)
    df, idx_f, w_f, pos_c, pos_f = _k2b_call(sig_c, ro, rd, dc_row, u_row)
    feats_f = _scg_call(table, idx_f.reshape(-1), w_f.reshape(-1))
    o_f = _k2a_call(feats_f, w1, b1r, w2, b2r)

    sd, sr, sg, sb, ss = _scm_call(
        pos_c.reshape(-1), pos_f.reshape(-1), df.reshape(-1), dc_row.reshape(S),
        o_c[:, 0], o_c[:, 1], o_c[:, 2], o_c[:, 3],
        o_f[:, 0], o_f[:, 1], o_f[:, 2], o_f[:, 3])

    rgb, dep, ws = _k4c_call(sd.reshape(NR, S2), sr.reshape(NR, S2),
                             sg.reshape(NR, S2), sb.reshape(NR, S2),
                             ss.reshape(NR, S2), dc_row)
    return (rgb.reshape(N_BATCH, NUM_RAYS, 3),
            dep.reshape(N_BATCH, NUM_RAYS, 1),
            ws.reshape(N_BATCH, NUM_RAYS, 1))


# speedup vs baseline: 2.3344x; 2.3344x over previous
import functools

import jax
import jax.numpy as jnp
import numpy as np
from jax import lax
from jax.experimental import pallas as pl
from jax.experimental.pallas import tpu as pltpu
from jax.experimental.pallas import tpu_sc as plsc

N_BATCH = 2
NUM_RAYS = 4096
DEPTH_RES = 48
N_IMPORTANCE = 48
RAY_START = 2.25
RAY_END = 3.3
C_FEAT = 32
HW = 256
HIDDEN = 64

NR = N_BATCH * NUM_RAYS
S = DEPTH_RES
M = NR * S
NTAP = 12
PLANE = HW * HW
S2 = 2 * S

_SU47 = np.triu(np.ones((47, 47), np.float32), 1)
_SU95 = np.triu(np.ones((95, 95), np.float32), 1)
_U45 = np.triu(np.ones((45, 45), np.float32), 0)

_RB = 512


def _softplus(x):
    return jnp.maximum(x, 0.0) + jnp.log1p(jnp.exp(-jnp.abs(x)))


def _taps(gx, gy, base, idx_out, w_out, p):
    x = (gx + 1.0) * (HW / 2.0) - 0.5
    y = (gy + 1.0) * (HW / 2.0) - 0.5
    x0 = jnp.floor(x)
    y0 = jnp.floor(y)
    wx1 = x - x0
    wx0 = 1.0 - wx1
    wy1 = y - y0
    wy0 = 1.0 - wy1
    taps = ((x0, y0, wx0 * wy0), (x0 + 1.0, y0, wx1 * wy0),
            (x0, y0 + 1.0, wx0 * wy1), (x0 + 1.0, y0 + 1.0, wx1 * wy1))
    for k, (xf, yf, wgt) in enumerate(taps):
        valid = ((xf >= 0) & (xf < HW) & (yf >= 0) & (yf < HW)).astype(jnp.float32)
        ix = jnp.clip(xf, 0.0, HW - 1.0).astype(jnp.int32)
        iy = jnp.clip(yf, 0.0, HW - 1.0).astype(jnp.int32)
        t = p * 4 + k
        idx_out[:, t * S:(t + 1) * S] = base + iy * HW + ix
        w_out[:, t * S:(t + 1) * S] = wgt * valid * jnp.float32(1.0 / 3.0)


def _emit_taps(ox, oy, oz, dx, dy, dz, depths, pid, idx_out, w_out):
    n = (pid * _RB) // NUM_RAYS
    base0 = n * 3 * PLANE
    cx = 2.0 * (ox + depths * dx)
    cy = 2.0 * (oy + depths * dy)
    cz = 2.0 * (oz + depths * dz)
    _taps(cx, cy, base0 + 0 * PLANE, idx_out, w_out, 0)
    _taps(cx, cz, base0 + 1 * PLANE, idx_out, w_out, 1)
    _taps(cz, cx, base0 + 2 * PLANE, idx_out, w_out, 2)


def _k1_body(ro_ref, rd_ref, dc_ref, idx_out, w_out):
    pid = pl.program_id(0)
    dc = dc_ref[0:1, :]
    _emit_taps(ro_ref[:, 0:1], ro_ref[:, 1:2], ro_ref[:, 2:3],
               rd_ref[:, 0:1], rd_ref[:, 1:2], rd_ref[:, 2:3],
               dc, pid, idx_out, w_out)


def _k1_call(ro, rd, dc_row):
    grid = NR // _RB
    return pl.pallas_call(
        _k1_body,
        grid=(grid,),
        in_specs=[pl.BlockSpec((_RB, 3), lambda i: (i, 0)),
                  pl.BlockSpec((_RB, 3), lambda i: (i, 0)),
                  pl.BlockSpec((1, S), lambda i: (0, 0))],
        out_specs=[pl.BlockSpec((_RB, NTAP * S), lambda i: (i, 0)),
                   pl.BlockSpec((_RB, NTAP * S), lambda i: (i, 0))],
        out_shape=[jax.ShapeDtypeStruct((NR, NTAP * S), jnp.int32),
                   jax.ShapeDtypeStruct((NR, NTAP * S), jnp.float32)],
    )(ro, rd, dc_row)


_MB = 4096


def _k2a_body(x_ref, w1_ref, b1_ref, w2_ref, b2_ref, o_ref):
    h = jnp.dot(x_ref[:, :], w1_ref[:, :], preferred_element_type=jnp.float32)
    h = _softplus(h + b1_ref[0:1, :])
    o = jnp.dot(h, w2_ref[:, :], preferred_element_type=jnp.float32) + b2_ref[0:1, :]
    sigma = o[:, 0:1]
    rgb = (1.0 / (1.0 + jnp.exp(-o[:, 1:4]))) * 1.002 - 0.001
    o_ref[:, :] = jnp.concatenate([sigma, rgb], axis=1)


def _k2a_call(feats, w1, b1r, w2, b2r):
    grid = M // _MB
    return pl.pallas_call(
        _k2a_body,
        grid=(grid,),
        in_specs=[pl.BlockSpec((_MB, C_FEAT), lambda i: (i, 0)),
                  pl.BlockSpec((C_FEAT, HIDDEN), lambda i: (0, 0)),
                  pl.BlockSpec((1, HIDDEN), lambda i: (0, 0)),
                  pl.BlockSpec((HIDDEN, 4), lambda i: (0, 0)),
                  pl.BlockSpec((1, 4), lambda i: (0, 0))],
        out_specs=pl.BlockSpec((_MB, 4), lambda i: (i, 0)),
        out_shape=jax.ShapeDtypeStruct((M, 4), jnp.float32),
    )(feats, w1, b1r, w2, b2r)


def _k2b_body(sig_ref, ro_ref, rd_ref, dc_ref, u_ref, su47_ref, u45_ref,
              df_out, idx_out, w_out, posc_out, posf_out):
    pid = pl.program_id(0)
    dc = dc_ref[0:1, :]
    sig = sig_ref[:, :]
    deltas = dc[:, 1:] - dc[:, :-1]
    dm = 0.5 * (sig[:, :-1] + sig[:, 1:])
    dd = _softplus(dm - 1.0) * deltas
    e = jnp.exp(-dd)
    alpha = 1.0 - e
    logt = jnp.log(e + 1e-10)
    texcl = jnp.exp(jnp.dot(logt, su47_ref[:, :], preferred_element_type=jnp.float32))
    w = alpha * texcl

    ninf = jnp.float32(-jnp.inf)
    left = jnp.concatenate([jnp.full((w.shape[0], 1), ninf), w], axis=1)
    right = jnp.concatenate([w, jnp.full((w.shape[0], 1), ninf)], axis=1)
    wmax = jnp.maximum(left, right)
    wavg = 0.5 * (wmax[:, :-1] + wmax[:, 1:]) + 0.01
    wpdf = wavg[:, 1:46] + 1e-5
    pdf = wpdf / jnp.sum(wpdf, axis=1, keepdims=True)
    cdf = jnp.dot(pdf, u45_ref[:, :], preferred_element_type=jnp.float32)
    u = u_ref[0:1, :]

    zmid = 0.5 * (dc[:, :-1] + dc[:, 1:])
    zero = jnp.zeros(sig.shape, jnp.float32)
    cdf_g0 = zero
    bins_g0 = zero + zmid[:, 0:1]
    cdf_g1 = zero + cdf[:, 44:45]
    bins_g1 = zero + zmid[:, 45:46]
    for j in range(45):
        m = cdf[:, j:j + 1] <= u
        cdf_g0 = jnp.where(m, cdf[:, j:j + 1], cdf_g0)
        bins_g0 = jnp.where(m, zmid[:, j + 1:j + 2], bins_g0)
    for j in range(44, -1, -1):
        m = cdf[:, j:j + 1] > u
        cdf_g1 = jnp.where(m, cdf[:, j:j + 1], cdf_g1)
        bins_g1 = jnp.where(m, zmid[:, j + 1:j + 2], bins_g1)
    denom = cdf_g1 - cdf_g0
    denom = jnp.where(denom < 1e-5, 1.0, denom)
    df = bins_g0 + (u - cdf_g0) / denom * (bins_g1 - bins_g0)
    df_out[:, :] = df

    iota = lax.broadcasted_iota(jnp.int32, sig.shape, 1)
    pc = iota
    pf = iota
    for j in range(S):
        pc = pc + (df[:, j:j + 1] < dc).astype(jnp.int32)
        pf = pf + (dc[:, j:j + 1] <= df).astype(jnp.int32)
    posc_out[:, :] = pc
    posf_out[:, :] = pf

    _emit_taps(ro_ref[:, 0:1], ro_ref[:, 1:2], ro_ref[:, 2:3],
               rd_ref[:, 0:1], rd_ref[:, 1:2], rd_ref[:, 2:3],
               df, pid, idx_out, w_out)


def _k2b_call(sig, ro, rd, dc_row, u_row):
    grid = NR // _RB
    return pl.pallas_call(
        _k2b_body,
        grid=(grid,),
        in_specs=[pl.BlockSpec((_RB, S), lambda i: (i, 0)),
                  pl.BlockSpec((_RB, 3), lambda i: (i, 0)),
                  pl.BlockSpec((_RB, 3), lambda i: (i, 0)),
                  pl.BlockSpec((1, S), lambda i: (0, 0)),
                  pl.BlockSpec((1, S), lambda i: (0, 0)),
                  pl.BlockSpec((47, 47), lambda i: (0, 0)),
                  pl.BlockSpec((45, 45), lambda i: (0, 0))],
        out_specs=[pl.BlockSpec((_RB, S), lambda i: (i, 0)),
                   pl.BlockSpec((_RB, NTAP * S), lambda i: (i, 0)),
                   pl.BlockSpec((_RB, NTAP * S), lambda i: (i, 0)),
                   pl.BlockSpec((_RB, S), lambda i: (i, 0)),
                   pl.BlockSpec((_RB, S), lambda i: (i, 0))],
        out_shape=[jax.ShapeDtypeStruct((NR, S), jnp.float32),
                   jax.ShapeDtypeStruct((NR, NTAP * S), jnp.int32),
                   jax.ShapeDtypeStruct((NR, NTAP * S), jnp.float32),
                   jax.ShapeDtypeStruct((NR, S), jnp.int32),
                   jax.ShapeDtypeStruct((NR, S), jnp.int32)],
    )(sig, ro, rd, dc_row, u_row, jnp.asarray(_SU47), jnp.asarray(_U45))


def _k4c_body(d_ref, r_ref, g_ref, b_ref, s_ref, dc_ref, su95_ref,
              rgb_out, dep_out, ws_out):
    d = d_ref[:, :]
    sg = s_ref[:, :]
    deltas = d[:, 1:] - d[:, :-1]
    dmid = 0.5 * (d[:, :-1] + d[:, 1:])
    smid = 0.5 * (sg[:, :-1] + sg[:, 1:])
    dd = _softplus(smid - 1.0) * deltas
    e = jnp.exp(-dd)
    alpha = 1.0 - e
    logt = jnp.log(e + 1e-10)
    texcl = jnp.exp(jnp.dot(logt, su95_ref[:, :], preferred_element_type=jnp.float32))
    w = alpha * texcl
    wtot = jnp.sum(w, axis=1, keepdims=True)
    outs = []
    for c_ref in (r_ref, g_ref, b_ref):
        c = c_ref[:, :]
        cmid = 0.5 * (c[:, :-1] + c[:, 1:])
        outs.append(jnp.sum(w * cmid, axis=1, keepdims=True))
    rgb = jnp.concatenate(outs, axis=1) * 2.0 - 1.0
    dep = jnp.sum(w * dmid, axis=1, keepdims=True) / wtot
    dep = jnp.where(dep != dep, jnp.float32(jnp.inf), dep)
    dmin = dc_ref[0:1, 0:1]
    dmax = dc_ref[0:1, S - 1:S]
    dep = jnp.minimum(jnp.maximum(dep, dmin), dmax)
    rgb_out[:, :] = rgb
    dep_out[:, :] = dep
    ws_out[:, :] = wtot


def _k4c_call(sd, sr, sg, sb, ss, dc_row):
    grid = NR // _RB
    return pl.pallas_call(
        _k4c_body,
        grid=(grid,),
        in_specs=[pl.BlockSpec((_RB, S2), lambda i: (i, 0)) for _ in range(5)] +
                 [pl.BlockSpec((1, S), lambda i: (0, 0)),
                  pl.BlockSpec((95, 95), lambda i: (0, 0))],
        out_specs=[pl.BlockSpec((_RB, 3), lambda i: (i, 0)),
                   pl.BlockSpec((_RB, 1), lambda i: (i, 0)),
                   pl.BlockSpec((_RB, 1), lambda i: (i, 0))],
        out_shape=[jax.ShapeDtypeStruct((NR, 3), jnp.float32),
                   jax.ShapeDtypeStruct((NR, 1), jnp.float32),
                   jax.ShapeDtypeStruct((NR, 1), jnp.float32)],
    )(sd, sr, sg, sb, ss, dc_row, jnp.asarray(_SU95))


_G = 96
_IDXC = _G * NTAP
_NW = 32
_SAMP_W = M // _NW
_CHUNKS = _SAMP_W // _G


def _scg_body(table_hbm, idx_hbm, w_hbm, out_hbm,
              idx_v0, idx_v1, w_v0, w_v1, rows_v0, rows_v1, out_v0, out_v1,
              sem_io, sem_g, sem_out):
    nc = 2
    wid = lax.axis_index("s") * nc + lax.axis_index("c")
    base_s = wid * _SAMP_W
    nsub = _IDXC // 128
    bufs = ((idx_v0, w_v0, rows_v0, out_v0), (idx_v1, w_v1, rows_v1, out_v1))

    def io_start(c, b):
        ib = (base_s + c * _G) * NTAP
        idx_v, w_v = bufs[b][0], bufs[b][1]
        pltpu.async_copy(idx_hbm.at[pl.ds(ib, _IDXC)], idx_v, sem_io)
        pltpu.async_copy(w_hbm.at[pl.ds(ib, _IDXC)], w_v.at[pl.ds(0, _IDXC)],
                         sem_io)

    def io_wait(c, b):
        ib = (base_s + c * _G) * NTAP
        idx_v, w_v = bufs[b][0], bufs[b][1]
        pltpu.make_async_copy(idx_hbm.at[pl.ds(ib, _IDXC)], idx_v, sem_io).wait()
        pltpu.make_async_copy(w_hbm.at[pl.ds(ib, _IDXC)],
                              w_v.at[pl.ds(0, _IDXC)], sem_io).wait()

    def g_start(b):
        idx_v, rows_v = bufs[b][0], bufs[b][2]
        for j in range(nsub):
            pltpu.async_copy(table_hbm.at[idx_v.at[pl.ds(j * 128, 128)]],
                             rows_v.at[pl.ds(j * 128, 128)], sem_g)

    def g_wait(b):
        idx_v, rows_v = bufs[b][0], bufs[b][2]
        for j in range(nsub):
            pltpu.make_async_copy(table_hbm.at[idx_v.at[pl.ds(j * 128, 128)]],
                                  rows_v.at[pl.ds(j * 128, 128)], sem_g).wait()

    def out_start(c, b):
        pltpu.async_copy(bufs[b][3], out_hbm.at[pl.ds(base_s + c * _G, _G), :],
                         sem_out)

    def out_wait(c, b):
        pltpu.make_async_copy(bufs[b][3],
                              out_hbm.at[pl.ds(base_s + c * _G, _G), :],
                              sem_out).wait()

    def compute(b):
        w_v, rows_v, out_v = bufs[b][1], bufs[b][2], bufs[b][3]

        def pair(h, _):
            for q in range(2):
                g = h * 2 + q
                lr = g // S
                sloc = g - lr * S
                w12 = w_v[pl.ds(g * NTAP, 16)]
                acc_lo = jnp.zeros((16,), jnp.float32)
                acc_hi = jnp.zeros((16,), jnp.float32)
                for t in range(NTAP):
                    flat = lr * (NTAP * S) + t * S + sloc
                    wv = lax.gather(
                        w12, jnp.full((16, 1), t, jnp.int32),
                        lax.GatherDimensionNumbers(offset_dims=(),
                                                   collapsed_slice_dims=(0,),
                                                   start_index_map=(0,)),
                        (1,), mode=lax.GatherScatterMode.PROMISE_IN_BOUNDS)
                    lo = rows_v[flat, pl.ds(0, 16)]
                    hi = rows_v[flat, pl.ds(16, 16)]
                    acc_lo = acc_lo + wv * lo
                    acc_hi = acc_hi + wv * hi
                out_v[g, pl.ds(0, 16)] = acc_lo
                out_v[g, pl.ds(16, 16)] = acc_hi
            return _

        lax.fori_loop(0, _G // 2, pair, 0)

    io_start(0, 0)
    io_wait(0, 0)
    g_start(0)
    io_start(1, 1)

    def step(cc, _):
        for b in (0, 1):
            c = cc * 2 + b
            io_wait(c + 1, 1 - b)
            g_start(1 - b)
            g_wait(b)
            lax.cond(c >= 2, lambda: out_wait(c - 2, b), lambda: None)
            compute(b)
            lax.cond(c + 2 < _CHUNKS, lambda: io_start(c + 2, b), lambda: None)
            out_start(c, b)
        return _

    lax.fori_loop(0, (_CHUNKS - 2) // 2, step, 0)
    c0 = _CHUNKS - 2
    for b in (0, 1):
        c = c0 + b
        if c + 1 < _CHUNKS:
            io_wait(c + 1, 1 - b)
            g_start(1 - b)
        g_wait(b)
        out_wait(c - 2, b)
        compute(b)
        out_start(c, b)
    out_wait(_CHUNKS - 2, 0)
    out_wait(_CHUNKS - 1, 1)


def _scg_call(table, idx_flat, w_flat):
    mesh = plsc.VectorSubcoreMesh(core_axis_name="c", subcore_axis_name="s")
    f = functools.partial(
        pl.kernel, mesh=mesh,
        compiler_params=pltpu.CompilerParams(use_tc_tiling_on_sc=False,
                                             needs_layout_passes=False),
        out_type=jax.ShapeDtypeStruct((M, C_FEAT), jnp.float32),
        scratch_types=[pltpu.VMEM((_IDXC,), jnp.int32),
                       pltpu.VMEM((_IDXC,), jnp.int32),
                       pltpu.VMEM((_IDXC + 16,), jnp.float32),
                       pltpu.VMEM((_IDXC + 16,), jnp.float32),
                       pltpu.VMEM((_IDXC, C_FEAT), jnp.float32),
                       pltpu.VMEM((_IDXC, C_FEAT), jnp.float32),
                       pltpu.VMEM((_G, C_FEAT), jnp.float32),
                       pltpu.VMEM((_G, C_FEAT), jnp.float32),
                       pltpu.SemaphoreType.DMA,
                       pltpu.SemaphoreType.DMA,
                       pltpu.SemaphoreType.DMA],
    )(_scg_body)
    return f(table, idx_flat, w_flat)


_CR = 32
_RAYS_W = NR // _NW
_MCHUNKS = _RAYS_W // _CR


def _scm_body(posc_hbm, posf_hbm, df_hbm, dc_hbm,
              sc_hbm, rc_hbm, gc_hbm, bc_hbm,
              sf_hbm, rf_hbm, gf_hbm, bf_hbm,
              od_hbm, or_hbm, og_hbm, ob_hbm, os_hbm,
              posc_v, posf_v, df_v, dc_v,
              sc_v, rc_v, gc_v, bc_v, sf_v, rf_v, gf_v, bf_v,
              od_v, or_v, og_v, ob_v, os_v):
    nc = 2
    wid = lax.axis_index("s") * nc + lax.axis_index("c")
    pltpu.sync_copy(dc_hbm, dc_v)

    def chunk(c, _):
        rbase = wid * _RAYS_W + c * _CR
        sb = rbase * S
        ns = _CR * S
        pltpu.sync_copy(posc_hbm.at[pl.ds(sb, ns)], posc_v)
        pltpu.sync_copy(posf_hbm.at[pl.ds(sb, ns)], posf_v)
        pltpu.sync_copy(df_hbm.at[pl.ds(sb, ns)], df_v)
        for src, dst in ((sc_hbm, sc_v), (rc_hbm, rc_v), (gc_hbm, gc_v),
                         (bc_hbm, bc_v), (sf_hbm, sf_v), (rf_hbm, rf_v),
                         (gf_hbm, gf_v), (bf_hbm, bf_v)):
            pltpu.sync_copy(src.at[pl.ds(sb, ns)], dst)

        def ray(r, _):
            rb96 = r * S2
            for g in range(S // 16):
                off = r * S + g * 16
                pc = posc_v[pl.ds(off, 16)] + rb96
                pf = posf_v[pl.ds(off, 16)] + rb96
                plsc.store_scatter(od_v, [pc], dc_v[pl.ds(g * 16, 16)])
                plsc.store_scatter(os_v, [pc], sc_v[pl.ds(off, 16)])
                plsc.store_scatter(or_v, [pc], rc_v[pl.ds(off, 16)])
                plsc.store_scatter(og_v, [pc], gc_v[pl.ds(off, 16)])
                plsc.store_scatter(ob_v, [pc], bc_v[pl.ds(off, 16)])
                plsc.store_scatter(od_v, [pf], df_v[pl.ds(off, 16)])
                plsc.store_scatter(os_v, [pf], sf_v[pl.ds(off, 16)])
                plsc.store_scatter(or_v, [pf], rf_v[pl.ds(off, 16)])
                plsc.store_scatter(og_v, [pf], gf_v[pl.ds(off, 16)])
                plsc.store_scatter(ob_v, [pf], bf_v[pl.ds(off, 16)])
            return _

        lax.fori_loop(0, _CR, ray, 0)
        ob = rbase * S2
        nso = _CR * S2
        pltpu.sync_copy(od_v, od_hbm.at[pl.ds(ob, nso)])
        pltpu.sync_copy(or_v, or_hbm.at[pl.ds(ob, nso)])
        pltpu.sync_copy(og_v, og_hbm.at[pl.ds(ob, nso)])
        pltpu.sync_copy(ob_v, ob_hbm.at[pl.ds(ob, nso)])
        pltpu.sync_copy(os_v, os_hbm.at[pl.ds(ob, nso)])
        return _

    lax.fori_loop(0, _MCHUNKS, chunk, 0)


def _scm_call(posc, posf, df, dc48, sc, rc, gc, bc, sf, rf, gf, bf):
    mesh = plsc.VectorSubcoreMesh(core_axis_name="c", subcore_axis_name="s")
    ns = _CR * S
    nso = _CR * S2
    f = functools.partial(
        pl.kernel, mesh=mesh,
        compiler_params=pltpu.CompilerParams(use_tc_tiling_on_sc=False,
                                             needs_layout_passes=False),
        out_type=[jax.ShapeDtypeStruct((NR * S2,), jnp.float32) for _ in range(5)],
        scratch_types=[pltpu.VMEM((ns,), jnp.int32),
                       pltpu.VMEM((ns,), jnp.int32),
                       pltpu.VMEM((ns,), jnp.float32),
                       pltpu.VMEM((S,), jnp.float32)] +
                      [pltpu.VMEM((ns,), jnp.float32) for _ in range(8)] +
                      [pltpu.VMEM((nso,), jnp.float32) for _ in range(5)],
    )(_scm_body)
    return f(posc, posf, df, dc48, sc, rc, gc, bc, sf, rf, gf, bf)


def kernel(planes, ray_origins, ray_directions, w1, b1, w2, b2):
    table = planes.transpose(0, 1, 3, 4, 2).reshape(N_BATCH * 3 * PLANE, C_FEAT)
    delta = (RAY_END - RAY_START) / (DEPTH_RES - 1)
    dc_row = (jnp.linspace(RAY_START, RAY_END, S, dtype=jnp.float32)
              + jnp.float32(0.5 * delta)).reshape(1, S)
    u_row = jnp.linspace(0.0, 1.0, N_IMPORTANCE, dtype=jnp.float32).reshape(1, S)
    ro = ray_origins.reshape(NR, 3)
    rd = ray_directions.reshape(NR, 3)
    b1r = b1.reshape(1, HIDDEN)
    b2r = b2.reshape(1, 4)

    def _wT(w):
        return w.reshape(NR, NTAP, S).transpose(0, 2, 1).reshape(-1)

    idx_c, w_c = _k1_call(ro, rd, dc_row)
    feats_c = _scg_call(table, idx_c.reshape(-1), _wT(w_c))
    o_c = _k2a_call(feats_c, w1, b1r, w2, b2r)

    sig_c = o_c[:, 0].reshape(NR, S)
    df, idx_f, w_f, pos_c, pos_f = _k2b_call(sig_c, ro, rd, dc_row, u_row)
    feats_f = _scg_call(table, idx_f.reshape(-1), _wT(w_f))
    o_f = _k2a_call(feats_f, w1, b1r, w2, b2r)

    sd, sr, sg, sb, ss = _scm_call(
        pos_c.reshape(-1), pos_f.reshape(-1), df.reshape(-1), dc_row.reshape(S),
        o_c[:, 0], o_c[:, 1], o_c[:, 2], o_c[:, 3],
        o_f[:, 0], o_f[:, 1], o_f[:, 2], o_f[:, 3])

    rgb, dep, ws = _k4c_call(sd.reshape(NR, S2), sr.reshape(NR, S2),
                             sg.reshape(NR, S2), sb.reshape(NR, S2),
                             ss.reshape(NR, S2), dc_row)
    return (rgb.reshape(N_BATCH, NUM_RAYS, 3),
            dep.reshape(N_BATCH, NUM_RAYS, 1),
            ws.reshape(N_BATCH, NUM_RAYS, 1))
